# baseline (speedup 1.0000x reference)
import jax
import jax.numpy as jnp
from jax import lax
from jax.experimental import pallas as pl
from jax.experimental.pallas import tpu as pltpu
from jax.experimental.pallas import tpu_sc as plsc

_TM = 2048


def _pair_body(pr_ref, pc_ref, mr_ref, mc_ref,
               adj0_ref, adj1_ref, adj2_ref, adj3_ref, pair_ref):
    i = pl.program_id(1)

    @pl.when(i == 0)
    def _init():
        pair_ref[...] = jnp.zeros_like(pair_ref)

    pr = pr_ref[0]
    pc = pc_ref[0]
    mr = mr_ref[0]
    mc = mc_ref[0]

    r2r = jnp.sum(pr * pr, axis=1, keepdims=True)
    r2c = jnp.sum(pc * pc, axis=0, keepdims=True)
    pr_aug = jnp.concatenate([pr, r2r, jnp.ones_like(r2r)], axis=1) * mr
    pc_aug = jnp.concatenate([-2.0 * pc, jnp.ones_like(r2c), r2c],
                             axis=0) * mc
    d2 = jax.lax.dot_general(pr_aug, pc_aug, (((1,), (0,)), ((), ())),
                             preferred_element_type=jnp.float32)
    d2 = jnp.maximum(d2, 0.0)
    dist = d2 * jax.lax.rsqrt(jnp.maximum(d2, 1e-35))
    q = dist.shape[1] // 4
    t = (adj0_ref[0].astype(jnp.float32) * dist[:, 0 * q:1 * q]
         + adj1_ref[0].astype(jnp.float32) * dist[:, 1 * q:2 * q]
         + adj2_ref[0].astype(jnp.float32) * dist[:, 2 * q:3 * q]
         + adj3_ref[0].astype(jnp.float32) * dist[:, 3 * q:4 * q])
    pair_ref[...] = pair_ref[...] + jnp.sum(t)


def _pair_energy(positions, pos_c, mask_row, mask_col, adjacency):
    B, N, _ = positions.shape
    TM = _TM
    grid = (B, N // TM)
    pair = pl.pallas_call(
        _pair_body,
        grid=grid,
        in_specs=[
            pl.BlockSpec((1, TM, 3), lambda b, i: (b, i, 0)),
            pl.BlockSpec((1, 3, N), lambda b, i: (b, 0, 0)),
            pl.BlockSpec((1, TM, 1), lambda b, i: (b, i, 0)),
            pl.BlockSpec((1, 1, N), lambda b, i: (b, 0, 0)),
            pl.BlockSpec((1, TM, N // 4), lambda b, i: (b, i, 0)),
            pl.BlockSpec((1, TM, N // 4), lambda b, i: (b, i, 1)),
            pl.BlockSpec((1, TM, N // 4), lambda b, i: (b, i, 2)),
            pl.BlockSpec((1, TM, N // 4), lambda b, i: (b, i, 3)),
        ],
        out_specs=pl.BlockSpec((1, 8, 128), lambda b, i: (b, 0, 0)),
        out_shape=jax.ShapeDtypeStruct((B, 8, 128), jnp.float32),
        compiler_params=pltpu.CompilerParams(
            dimension_semantics=("parallel", "arbitrary")),
    )(positions, pos_c, mask_row, mask_col,
      adjacency, adjacency, adjacency, adjacency)
    return pair[:, 0, 0]


def _atom_energy(idx_flat, se_pad, total):
    info = plsc.get_sparse_core_info()
    NC, NS, L = info.num_cores, info.num_subcores, info.num_lanes
    NW = NC * NS
    per_w = total // NW

    def _body(idx_hbm, se_hbm, out_hbm, idx_v, se_v, acc_v):
        wid = lax.axis_index("s") * NC + lax.axis_index("c")
        base = wid * per_w
        pltpu.sync_copy(idx_hbm.at[pl.ds(base, per_w)], idx_v)
        pltpu.sync_copy(se_hbm, se_v)
        acc = jnp.zeros((L,), jnp.float32)
        for i in range(per_w // L):
            iv = idx_v[pl.ds(i * L, L)]
            acc = acc + plsc.load_gather(se_v, [iv])
        acc_v[...] = acc
        pltpu.sync_copy(acc_v, out_hbm.at[wid])

    out = pl.kernel(
        _body,
        mesh=plsc.VectorSubcoreMesh(core_axis_name="c", subcore_axis_name="s"),
        out_type=jax.ShapeDtypeStruct((NW, L), jnp.float32),
        scratch_types=[
            pltpu.VMEM((per_w,), jnp.int32),
            pltpu.VMEM((128,), jnp.float32),
            pltpu.VMEM((L,), jnp.float32),
        ],
        compiler_params=pltpu.CompilerParams(needs_layout_passes=False),
    )(idx_flat, se_pad)
    return out


def kernel(node_indices, positions, adjacency, mask, species_energy,
           pair_weight):
    B, N = node_indices.shape
    S = species_energy.shape[0]

    maskf = mask.astype(jnp.float32)
    mask_row = maskf.reshape(B, N, 1)
    mask_col = maskf.reshape(B, 1, N)
    pos_c = positions.transpose(0, 2, 1)
    idx_flat = jnp.where(mask, node_indices, 127).astype(jnp.int32)
    idx_flat = idx_flat.reshape(B * N)
    se_pad = jnp.zeros((128,), jnp.float32).at[:S].set(species_energy)

    pair = _pair_energy(positions, pos_c, mask_row, mask_col, adjacency)
    part = _atom_energy(idx_flat, se_pad, B * N)
    atom = jnp.sum(part.reshape(B, -1), axis=1)

    return atom + pair_weight * pair


# speedup vs baseline: 1.0001x; 1.0001x over previous
import jax
import jax.numpy as jnp
from jax import lax
from jax.experimental import pallas as pl
from jax.experimental.pallas import tpu as pltpu
from jax.experimental.pallas import tpu_sc as plsc

_TM = 2048


def _pair_body(pr_ref, pc_ref, mr_ref, mc_ref,
               adj0_ref, adj1_ref, adj2_ref, adj3_ref, pair_ref):
    i = pl.program_id(1)

    @pl.when(i == 0)
    def _init():
        pair_ref[...] = jnp.zeros_like(pair_ref)

    pr = pr_ref[0]
    pc = pc_ref[0]
    mr = mr_ref[0]
    mc = mc_ref[0]

    r2r = jnp.sum(pr * pr, axis=1, keepdims=True)
    r2c = jnp.sum(pc * pc, axis=0, keepdims=True)
    pr_aug = jnp.concatenate([pr, r2r, jnp.ones_like(r2r)], axis=1) * mr
    pc_aug = jnp.concatenate([-2.0 * pc, jnp.ones_like(r2c), r2c],
                             axis=0) * mc
    d2 = jax.lax.dot_general(pr_aug, pc_aug, (((1,), (0,)), ((), ())),
                             preferred_element_type=jnp.float32)
    d2 = jnp.maximum(d2, 0.0)
    dist = d2 * jax.lax.rsqrt(jnp.maximum(d2, 1e-35))
    q = dist.shape[1] // 4
    t = (adj0_ref[0].astype(jnp.float32) * dist[:, 0 * q:1 * q]
         + adj1_ref[0].astype(jnp.float32) * dist[:, 1 * q:2 * q]
         + adj2_ref[0].astype(jnp.float32) * dist[:, 2 * q:3 * q]
         + adj3_ref[0].astype(jnp.float32) * dist[:, 3 * q:4 * q])
    pair_ref[...] = pair_ref[...] + jnp.sum(t)


def _pair_energy(positions, pos_c, mask_row, mask_col, adjacency):
    B, N, _ = positions.shape
    TM = _TM
    grid = (B, N // TM)
    pair = pl.pallas_call(
        _pair_body,
        grid=grid,
        in_specs=[
            pl.BlockSpec((1, TM, 3), lambda b, i: (b, i, 0)),
            pl.BlockSpec((1, 3, N), lambda b, i: (b, 0, 0)),
            pl.BlockSpec((1, TM, 1), lambda b, i: (b, i, 0)),
            pl.BlockSpec((1, 1, N), lambda b, i: (b, 0, 0)),
            pl.BlockSpec((1, TM, N // 4), lambda b, i: (b, i, 0)),
            pl.BlockSpec((1, TM, N // 4), lambda b, i: (b, i, 1)),
            pl.BlockSpec((1, TM, N // 4), lambda b, i: (b, i, 2)),
            pl.BlockSpec((1, TM, N // 4), lambda b, i: (b, i, 3)),
        ],
        out_specs=pl.BlockSpec((1, 8, 128), lambda b, i: (b, 0, 0)),
        out_shape=jax.ShapeDtypeStruct((B, 8, 128), jnp.float32),
        compiler_params=pltpu.CompilerParams(
            dimension_semantics=("parallel", "arbitrary")),
    )(positions, pos_c, mask_row, mask_col,
      adjacency, adjacency, adjacency, adjacency)
    return pair[:, 0, 0]


def _atom_energy(idx_flat, se_pad, total):
    info = plsc.get_sparse_core_info()
    NC, NS, L = info.num_cores, info.num_subcores, info.num_lanes
    NW = NC * NS
    per_w = total // NW

    def _body(idx_hbm, se_hbm, out_hbm, idx_v, se_v, acc_v):
        wid = lax.axis_index("s") * NC + lax.axis_index("c")
        base = wid * per_w
        pltpu.sync_copy(idx_hbm.at[pl.ds(base, per_w)], idx_v)
        pltpu.sync_copy(se_hbm, se_v)
        acc = jnp.zeros((L,), jnp.float32)
        for i in range(per_w // L):
            iv = idx_v[pl.ds(i * L, L)]
            acc = acc + plsc.load_gather(se_v, [iv])
        acc_v[...] = acc
        pltpu.sync_copy(acc_v, out_hbm.at[wid])

    out = pl.kernel(
        _body,
        mesh=plsc.VectorSubcoreMesh(core_axis_name="c", subcore_axis_name="s"),
        out_type=jax.ShapeDtypeStruct((NW, L), jnp.float32),
        scratch_types=[
            pltpu.VMEM((per_w,), jnp.int32),
            pltpu.VMEM((128,), jnp.float32),
            pltpu.VMEM((L,), jnp.float32),
        ],
        compiler_params=pltpu.CompilerParams(needs_layout_passes=False),
    )(idx_flat, se_pad)
    return out


def kernel(node_indices, positions, adjacency, mask, species_energy,
           pair_weight):
    B, N = node_indices.shape
    S = species_energy.shape[0]

    maskf = mask.astype(jnp.float32)
    mask_row = maskf.reshape(B, N, 1)
    mask_col = maskf.reshape(B, 1, N)
    pos_c = positions.transpose(0, 2, 1)
    idx_flat = jnp.where(mask, node_indices, 127).astype(jnp.int32)
    idx_flat = idx_flat.reshape(B * N)
    se_pad = jnp.zeros((128,), jnp.float32).at[:S].set(species_energy)

    part = _atom_energy(idx_flat, se_pad, B * N)
    pair = _pair_energy(positions, pos_c, mask_row, mask_col, adjacency)
    atom = jnp.sum(part.reshape(B, -1), axis=1)

    return atom + pair_weight * pair


# speedup vs baseline: 1.1083x; 1.1082x over previous
import jax
import jax.numpy as jnp
from jax.experimental import pallas as pl
from jax.experimental.pallas import tpu as pltpu


def _energy_body(idx_ref, pos_ref, m_ref, se_ref,
                 adj0_ref, adj1_ref, adj2_ref, adj3_ref,
                 pair_ref, atom_ref):
    p = pos_ref[0]
    m = m_ref[0]

    r2 = jnp.sum(p * p, axis=1, keepdims=True)
    one = jnp.ones_like(r2)
    row_aug = jnp.concatenate([p, r2, one], axis=1) * m
    col_aug = jnp.concatenate([-2.0 * p, one, r2], axis=1) * m
    d2 = jax.lax.dot_general(row_aug, col_aug, (((1,), (1,)), ((), ())),
                             preferred_element_type=jnp.float32)
    d2 = jnp.maximum(d2, 0.0)
    dist = d2 * jax.lax.rsqrt(jnp.maximum(d2, 1e-35))
    q = dist.shape[1] // 4
    t = (adj0_ref[0].astype(jnp.float32) * dist[:, 0 * q:1 * q]
         + adj1_ref[0].astype(jnp.float32) * dist[:, 1 * q:2 * q]
         + adj2_ref[0].astype(jnp.float32) * dist[:, 2 * q:3 * q]
         + adj3_ref[0].astype(jnp.float32) * dist[:, 3 * q:4 * q])
    pair_ref[...] = jnp.zeros_like(pair_ref) + jnp.sum(t)

    onehot = (jax.lax.broadcasted_iota(jnp.int32, (idx_ref.shape[1], 128), 1)
              == idx_ref[0]).astype(jnp.float32)
    ae = jnp.dot(onehot, se_ref[...], preferred_element_type=jnp.float32)
    atom_ref[...] = jnp.zeros_like(atom_ref) + jnp.sum(ae)


def kernel(node_indices, positions, adjacency, mask, species_energy,
           pair_weight):
    B, N = node_indices.shape
    S = species_energy.shape[0]

    m3 = mask.astype(jnp.float32).reshape(B, N, 1)
    idx2 = jnp.where(mask, node_indices, 127).astype(jnp.int32)
    idx2 = idx2.reshape(B, N, 1)
    se = jnp.zeros((128, 1), jnp.float32).at[:S, 0].set(species_energy)

    grid = (B,)
    pair, atom = pl.pallas_call(
        _energy_body,
        grid=grid,
        in_specs=[
            pl.BlockSpec((1, N, 1), lambda b: (b, 0, 0)),
            pl.BlockSpec((1, N, 3), lambda b: (b, 0, 0)),
            pl.BlockSpec((1, N, 1), lambda b: (b, 0, 0)),
            pl.BlockSpec((128, 1), lambda b: (0, 0)),
            pl.BlockSpec((1, N, N // 4), lambda b: (b, 0, 0)),
            pl.BlockSpec((1, N, N // 4), lambda b: (b, 0, 1)),
            pl.BlockSpec((1, N, N // 4), lambda b: (b, 0, 2)),
            pl.BlockSpec((1, N, N // 4), lambda b: (b, 0, 3)),
        ],
        out_specs=[
            pl.BlockSpec((1, 8, 128), lambda b: (b, 0, 0)),
            pl.BlockSpec((1, 8, 128), lambda b: (b, 0, 0)),
        ],
        out_shape=[
            jax.ShapeDtypeStruct((B, 8, 128), jnp.float32),
            jax.ShapeDtypeStruct((B, 8, 128), jnp.float32),
        ],
        compiler_params=pltpu.CompilerParams(
            dimension_semantics=("parallel",)),
    )(idx2, positions, m3, se, adjacency, adjacency, adjacency, adjacency)

    return atom[:, 0, 0] + pair_weight * pair[:, 0, 0]


# speedup vs baseline: 1.1681x; 1.0540x over previous
import jax
import jax.numpy as jnp
from jax.experimental import pallas as pl
from jax.experimental.pallas import tpu as pltpu

_TM = 2048


def _energy_body(idx_ref, pr_ref, pc_ref, mr_ref, mc_ref, se_ref,
                 *rest):
    adj_refs = rest[:16]
    pair_ref, atom_ref = rest[16], rest[17]
    i = pl.program_id(1)

    @pl.when(i == 0)
    def _init():
        pair_ref[...] = jnp.zeros_like(pair_ref)
        atom_ref[...] = jnp.zeros_like(atom_ref)

    pr = pr_ref[0]
    pc = pc_ref[0]
    mr = mr_ref[0]
    mc = mc_ref[0]

    r2r = jnp.sum(pr * pr, axis=1, keepdims=True)
    r2c = jnp.sum(pc * pc, axis=0, keepdims=True)
    pr_aug = jnp.concatenate([pr, r2r, jnp.ones_like(r2r)], axis=1) * mr
    pc_aug = jnp.concatenate([-2.0 * pc, jnp.ones_like(r2c), r2c],
                             axis=0) * mc
    d2 = jax.lax.dot_general(pr_aug, pc_aug, (((1,), (0,)), ((), ())),
                             preferred_element_type=jnp.float32)
    d2 = jnp.maximum(d2, 0.0)
    dist = d2 * jax.lax.rsqrt(jnp.maximum(d2, 1e-35))
    q = dist.shape[1] // 16
    t = sum(r[0].astype(jnp.float32) * dist[:, k * q:(k + 1) * q]
            for k, r in enumerate(adj_refs))
    pair_ref[...] = pair_ref[...] + jnp.sum(t)

    onehot = (jax.lax.broadcasted_iota(jnp.int32, (idx_ref.shape[1], 128), 1)
              == idx_ref[0]).astype(jnp.float32)
    ae = jnp.dot(onehot, se_ref[...], preferred_element_type=jnp.float32)
    atom_ref[...] = atom_ref[...] + jnp.sum(ae)


def kernel(node_indices, positions, adjacency, mask, species_energy,
           pair_weight):
    B, N = node_indices.shape
    S = species_energy.shape[0]
    TM = _TM

    maskf = mask.astype(jnp.float32)
    mask_row = maskf.reshape(B, N, 1)
    mask_col = maskf.reshape(B, 1, N)
    pos_c = positions.transpose(0, 2, 1)
    idx2 = jnp.where(mask, node_indices, 127).astype(jnp.int32)
    idx2 = idx2.reshape(B, N, 1)
    se = jnp.zeros((128, 1), jnp.float32).at[:S, 0].set(species_energy)

    grid = (B, N // TM)
    pair, atom = pl.pallas_call(
        _energy_body,
        grid=grid,
        in_specs=[
            pl.BlockSpec((1, TM, 1), lambda b, i: (b, i, 0)),
            pl.BlockSpec((1, TM, 3), lambda b, i: (b, i, 0)),
            pl.BlockSpec((1, 3, N), lambda b, i: (b, 0, 0)),
            pl.BlockSpec((1, TM, 1), lambda b, i: (b, i, 0)),
            pl.BlockSpec((1, 1, N), lambda b, i: (b, 0, 0)),
            pl.BlockSpec((128, 1), lambda b, i: (0, 0)),
            pl.BlockSpec((1, TM, N // 16), lambda b, i, k=0: (b, i, k)),
            pl.BlockSpec((1, TM, N // 16), lambda b, i, k=1: (b, i, k)),
            pl.BlockSpec((1, TM, N // 16), lambda b, i, k=2: (b, i, k)),
            pl.BlockSpec((1, TM, N // 16), lambda b, i, k=3: (b, i, k)),
            pl.BlockSpec((1, TM, N // 16), lambda b, i, k=4: (b, i, k)),
            pl.BlockSpec((1, TM, N // 16), lambda b, i, k=5: (b, i, k)),
            pl.BlockSpec((1, TM, N // 16), lambda b, i, k=6: (b, i, k)),
            pl.BlockSpec((1, TM, N // 16), lambda b, i, k=7: (b, i, k)),
            pl.BlockSpec((1, TM, N // 16), lambda b, i, k=8: (b, i, k)),
            pl.BlockSpec((1, TM, N // 16), lambda b, i, k=9: (b, i, k)),
            pl.BlockSpec((1, TM, N // 16), lambda b, i, k=10: (b, i, k)),
            pl.BlockSpec((1, TM, N // 16), lambda b, i, k=11: (b, i, k)),
            pl.BlockSpec((1, TM, N // 16), lambda b, i, k=12: (b, i, k)),
            pl.BlockSpec((1, TM, N // 16), lambda b, i, k=13: (b, i, k)),
            pl.BlockSpec((1, TM, N // 16), lambda b, i, k=14: (b, i, k)),
            pl.BlockSpec((1, TM, N // 16), lambda b, i, k=15: (b, i, k)),
        ],
        out_specs=[
            pl.BlockSpec((1, 8, 128), lambda b, i: (b, 0, 0)),
            pl.BlockSpec((1, 8, 128), lambda b, i: (b, 0, 0)),
        ],
        out_shape=[
            jax.ShapeDtypeStruct((B, 8, 128), jnp.float32),
            jax.ShapeDtypeStruct((B, 8, 128), jnp.float32),
        ],
        compiler_params=pltpu.CompilerParams(
            dimension_semantics=("parallel", "arbitrary")),
    )(idx2, positions, pos_c, mask_row, mask_col, se,
      *([adjacency] * 16))

    return atom[:, 0, 0] + pair_weight * pair[:, 0, 0]
